# baseline (speedup 1.0000x reference)
import functools

import jax
import jax.numpy as jnp
from jax.experimental import pallas as pl
from jax.experimental.pallas import tpu as pltpu

N, SIZE1, SIZE2, OUT_SIZE, NNZ, C = 8192, 64, 64, 64, 128, 32
BLOCK_N = 256


def _body(i1_ref, i2_ref, io_ref, s_ref, x_ref, y_ref, o_ref):
    o_ref[...] = jnp.zeros_like(o_ref)

    def step(k, _):
        a = x_ref[:, i1_ref[k], :]
        b = y_ref[:, i2_ref[k], :]
        inter = a * b * s_ref[k]
        io = io_ref[k]
        o_ref[:, io, :] = o_ref[:, io, :] + inter
        return 0

    jax.lax.fori_loop(0, NNZ, step, 0)


@jax.jit
def kernel(x, y, scale, index1, index2, index_out):
    grid = (N // BLOCK_N,)
    return pl.pallas_call(
        _body,
        grid=grid,
        in_specs=[
            pl.BlockSpec(memory_space=pltpu.SMEM),
            pl.BlockSpec(memory_space=pltpu.SMEM),
            pl.BlockSpec(memory_space=pltpu.SMEM),
            pl.BlockSpec(memory_space=pltpu.SMEM),
            pl.BlockSpec((BLOCK_N, SIZE1, C), lambda i: (i, 0, 0)),
            pl.BlockSpec((BLOCK_N, SIZE2, C), lambda i: (i, 0, 0)),
        ],
        out_specs=pl.BlockSpec((BLOCK_N, OUT_SIZE, C), lambda i: (i, 0, 0)),
        out_shape=jax.ShapeDtypeStruct((N, OUT_SIZE, C), x.dtype),
    )(index1, index2, index_out, scale, x, y)
